# baseline (speedup 1.0000x reference)
import functools

import jax
import jax.numpy as jnp
from jax import lax
from jax.experimental import pallas as pl
from jax.experimental.pallas import tpu as pltpu
from jax.experimental.pallas import tpu_sc as plsc


def kernel(x, pe_table):
    B, S, D = x.shape
    info = plsc.get_sparse_core_info()
    NW = info.num_cores * info.num_subcores
    rows_per_w = S // NW
    CH = 32
    NCHUNK = rows_per_w // CH

    mesh = plsc.VectorSubcoreMesh(core_axis_name="c", subcore_axis_name="s")

    @functools.partial(
        pl.kernel,
        out_type=jax.ShapeDtypeStruct((B, S, D), pe_table.dtype),
        mesh=mesh,
        scratch_types=[
            pltpu.VMEM((CH, D), pe_table.dtype),
        ],
    )
    def sc_copy(pe_hbm, out_hbm, buf):
        wid = lax.axis_index("s") * info.num_cores + lax.axis_index("c")
        base = wid * rows_per_w
        for ci in range(NCHUNK):
            r0 = base + ci * CH
            pltpu.sync_copy(pe_hbm.at[pl.ds(r0, CH)], buf)
            for b in range(B):
                pltpu.sync_copy(buf, out_hbm.at[b, pl.ds(r0, CH)])

    return sc_copy(pe_table[:S])


# speedup vs baseline: 1.0340x; 1.0340x over previous
import functools

import jax
import jax.numpy as jnp
from jax import lax
from jax.experimental import pallas as pl
from jax.experimental.pallas import tpu as pltpu
from jax.experimental.pallas import tpu_sc as plsc


def kernel(x, pe_table):
    B, S, D = x.shape
    info = plsc.get_sparse_core_info()
    NW = info.num_cores * info.num_subcores
    rows_per_w = S // NW
    CH = 32
    NCHUNK = rows_per_w // CH

    mesh = plsc.VectorSubcoreMesh(core_axis_name="c", subcore_axis_name="s")

    @functools.partial(
        pl.kernel,
        out_type=jax.ShapeDtypeStruct((B, S, D), pe_table.dtype),
        mesh=mesh,
        scratch_types=[
            pltpu.VMEM((CH, D), pe_table.dtype),
            pltpu.VMEM((CH, D), pe_table.dtype),
            pltpu.SemaphoreType.DMA,
            pltpu.SemaphoreType.DMA,
            pltpu.SemaphoreType.DMA,
            pltpu.SemaphoreType.DMA,
        ],
    )
    def sc_copy(pe_hbm, out_hbm, buf0, buf1, isem0, isem1, osem0, osem1):
        wid = lax.axis_index("s") * info.num_cores + lax.axis_index("c")
        base = wid * rows_per_w
        bufs = (buf0, buf1)
        isems = (isem0, isem1)
        osems = (osem0, osem1)

        def in_cp(ci):
            return pltpu.make_async_copy(
                pe_hbm.at[pl.ds(base + ci * CH, CH)], bufs[ci % 2], isems[ci % 2]
            )

        def out_cp(ci, b):
            return pltpu.make_async_copy(
                bufs[ci % 2], out_hbm.at[b, pl.ds(base + ci * CH, CH)], osems[ci % 2]
            )

        in_cp(0).start()
        for ci in range(NCHUNK):
            in_cp(ci).wait()
            if ci >= 1:
                for b in range(B):
                    out_cp(ci - 1, b).wait()
            if ci + 1 < NCHUNK:
                in_cp(ci + 1).start()
            for b in range(B):
                out_cp(ci, b).start()
        for b in range(B):
            out_cp(NCHUNK - 1, b).wait()

    return sc_copy(pe_table[:S])
